# baseline (speedup 1.0000x reference)
import functools

import jax
import jax.numpy as jnp
import numpy as np
from jax import lax
from jax.experimental import pallas as pl
from jax.experimental.pallas import tpu as pltpu
from jax.experimental.pallas import tpu_sc as plsc

_VP = 1024
_L = 16
_NW = 32
_NNEG = 5
_GCH = 128


def _gram_body(ab_hbm, o_ref, a_v, b_v, sem):
    v = ab_hbm.shape[0] // 2
    ca = pltpu.make_async_copy(ab_hbm.at[pl.ds(0, v)], a_v, sem)
    cb = pltpu.make_async_copy(ab_hbm.at[pl.ds(v, v)], b_v, sem)
    ca.start()
    cb.start()
    ca.wait()
    cb.wait()
    a = jnp.pad(a_v[...], ((0, _VP - v), (0, 0)))
    b = jnp.pad(b_v[...], ((0, _VP - v), (0, 0)))
    g = lax.dot_general(a, b, (((1,), (1,)), ((), ())),
                        preferred_element_type=jnp.float32)
    o_ref[...] = g.reshape(_VP * _VP // 128, 128)


def _log_sigmoid(x):
    u = jnp.exp(-jnp.abs(x))
    s = u / (2.0 + u)
    s2 = s * s
    p = 1.0 + s2 * (1.0 / 3.0 + s2 * (1.0 / 5.0 + s2 * (1.0 / 7.0)))
    return jnp.minimum(x, 0.0) - 2.0 * s * p


def _rotl(x, d):
    d = np.uint32(d)
    return (x << d) | (x >> np.uint32(32 - d))


def _threefry2x32(k1, k2, x0, x1):
    rots = ((13, 15, 26, 6), (17, 29, 16, 24))
    ks = (np.uint32(k1), np.uint32(k2),
          np.uint32(k1) ^ np.uint32(k2) ^ np.uint32(0x1BD11BDA))
    x0 = (x0 + ks[0]).astype(np.uint32)
    x1 = (x1 + ks[1]).astype(np.uint32)
    for i in range(5):
        for r in rots[i % 2]:
            x0 = (x0 + x1).astype(np.uint32)
            x1 = x0 ^ _rotl(x1, r)
        x0 = (x0 + ks[(i + 1) % 3]).astype(np.uint32)
        x1 = (x1 + ks[(i + 2) % 3] + np.uint32(i + 1)).astype(np.uint32)
    return x0, x1


def _np_randint(k1, k2, n, span):
    hi2 = np.zeros(2, np.uint32)
    lo2 = np.arange(2, dtype=np.uint32)
    b1, b2 = _threefry2x32(k1, k2, hi2, lo2)
    idx = np.arange(n, dtype=np.uint64)
    chi = (idx >> np.uint64(32)).astype(np.uint32)
    clo = (idx & np.uint64(0xFFFFFFFF)).astype(np.uint32)
    h1, h2 = _threefry2x32(b1[0], b2[0], chi, clo)
    l1, l2 = _threefry2x32(b1[1], b2[1], chi, clo)
    higher, lower = h1 ^ h2, l1 ^ l2
    span = np.uint32(span)
    mult = np.uint32((int(2**16 % span) ** 2) % int(span))
    return (((higher % span) * mult + lower % span) % span).astype(np.int32)


@functools.cache
def _neg_draw(B, V):
    neg = _np_randint(np.uint32(0), np.uint32(42), B * _NNEG, V).reshape(B, _NNEG)
    f0, f1 = _threefry2x32(np.uint32(0), np.uint32(42),
                           np.zeros(1, np.uint32), np.ones(1, np.uint32))
    res = _np_randint(f0[0], f1[0], B * _NNEG, V).reshape(B, _NNEG)
    bpw = B // _NW
    neg_wj = neg.reshape(_NW, bpw, _NNEG).transpose(0, 2, 1).reshape(-1)
    res_wj = res.reshape(_NW, bpw, _NNEG).transpose(0, 2, 1).reshape(-1)
    return np.ascontiguousarray(neg_wj | (res_wj << 10)).astype(np.int32)


@functools.cache
def _make_sc_kernel(B):
    bpw = B // _NW
    npw = bpw * _NNEG
    chunks = bpw // _L
    scale = -1.0 / B
    mesh = plsc.VectorSubcoreMesh(core_axis_name="c", subcore_axis_name="s")

    @functools.partial(
        pl.kernel, mesh=mesh,
        out_type=jax.ShapeDtypeStruct((_NW, _L), jnp.float32),
        scratch_types=[
            pltpu.VMEM((bpw,), jnp.int32),
            pltpu.VMEM((bpw,), jnp.int32),
            pltpu.VMEM((npw,), jnp.int32),
            pltpu.VMEM((_L,), jnp.int32),
            pltpu.VMEM((bpw,), jnp.int32),
            pltpu.VMEM((npw,), jnp.int32),
            pltpu.VMEM((bpw,), jnp.float32),
            pltpu.VMEM((npw,), jnp.float32),
            pltpu.VMEM((_L,), jnp.float32),
            pltpu.SemaphoreType.DMA,
            pltpu.SemaphoreType.DMA,
            pltpu.SemaphoreType.DMA,
            pltpu.SemaphoreType.DMA,
        ] + [pltpu.SemaphoreType.DMA] * _NNEG,
    )
    def sc_kernel(g_hbm, tgt_hbm, ctx_hbm, pk_hbm, nn_hbm, out_hbm,
                  tgt_v, ctx_v, pk_v, nn_v, pidx_v, nidx_v,
                  ps_v, ns_v, acc_v, semi, semk, semn, semp, *semj):
        info = plsc.get_sparse_core_info()
        wid = lax.axis_index("c") * info.num_subcores + lax.axis_index("s")
        eb = wid * bpw
        nb = wid * npw
        ci_t = pltpu.async_copy(tgt_hbm.at[pl.ds(eb, bpw)], tgt_v, semi)
        ci_c = pltpu.async_copy(ctx_hbm.at[pl.ds(eb, bpw)], ctx_v, semi)
        ci_p = pltpu.async_copy(pk_hbm.at[pl.ds(nb, npw)], pk_v, semk)
        ci_n = pltpu.async_copy(nn_hbm, nn_v, semn)
        ci_t.wait()
        ci_c.wait()

        def pos_idx_body(c, _):
            t16 = tgt_v[pl.ds(c * _L, _L)]
            c16 = ctx_v[pl.ds(c * _L, _L)]
            pidx_v[pl.ds(c * _L, _L)] = t16 * _VP + c16
            return 0
        lax.fori_loop(0, chunks, pos_idx_body, 0)
        pos_handles = [
            pltpu.async_copy(g_hbm.at[pidx_v.at[pl.ds(k * _GCH, _GCH)]],
                             ps_v.at[pl.ds(k * _GCH, _GCH)], semp)
            for k in range(bpw // _GCH)]

        ci_p.wait()
        neg_handles = []
        for j in range(_NNEG):
            def neg_idx_body(c, _, j=j):
                t16 = tgt_v[pl.ds(c * _L, _L)]
                c16 = ctx_v[pl.ds(c * _L, _L)]
                w16 = pk_v[pl.ds(j * bpw + c * _L, _L)]
                n16 = lax.bitwise_and(w16, _VP - 1)
                r16 = lax.shift_right_logical(w16, 10)
                n16 = jnp.where(n16 == c16, r16, n16)
                nidx_v[pl.ds(j * bpw + c * _L, _L)] = t16 * _VP + n16
                return 0
            lax.fori_loop(0, chunks, neg_idx_body, 0)
            neg_handles.append([
                pltpu.async_copy(
                    g_hbm.at[nidx_v.at[pl.ds(j * bpw + k * _GCH, _GCH)]],
                    ns_v.at[pl.ds(j * bpw + k * _GCH, _GCH)], semj[j])
                for k in range(bpw // _GCH)])

        for h in pos_handles:
            h.wait()

        def pos_acc_body(c, acc):
            x = ps_v[pl.ds(c * _L, _L)]
            return acc + _log_sigmoid(x)
        acc = lax.fori_loop(0, chunks, pos_acc_body,
                            jnp.zeros((_L,), jnp.float32), unroll=4)

        ci_n.wait()
        nn16 = nn_v[...]
        for j in range(_NNEG):
            for h in neg_handles[j]:
                h.wait()

            def neg_acc_body(c, jacc, j=j):
                x = ns_v[pl.ds(j * bpw + c * _L, _L)]
                return jacc + _log_sigmoid(-x)
            jacc = lax.fori_loop(0, chunks, neg_acc_body,
                                 jnp.zeros((_L,), jnp.float32), unroll=4)
            acc = acc + jnp.where(nn16 > j, jacc, jnp.zeros((_L,), jnp.float32))

        acc_v[...] = acc * scale
        pltpu.sync_copy(acc_v, out_hbm.at[wid])

    return sc_kernel


def kernel(in_table, out_table, target, context, num_neg):
    V, _ = in_table.shape
    B = target.shape[0]
    ab = jnp.concatenate(
        [in_table.astype(jnp.bfloat16), out_table.astype(jnp.bfloat16)], 0)
    gram = pl.pallas_call(
        _gram_body,
        in_specs=[pl.BlockSpec(memory_space=pltpu.MemorySpace.HBM)],
        out_shape=jax.ShapeDtypeStruct((_VP * _VP // 128, 128), jnp.float32),
        scratch_shapes=[
            pltpu.VMEM(in_table.shape, jnp.bfloat16),
            pltpu.VMEM(out_table.shape, jnp.bfloat16),
            pltpu.SemaphoreType.DMA,
        ],
    )(ab)
    gflat = gram.reshape(_VP * _VP)

    packed = _neg_draw(B, V)
    nn16 = jnp.full((_L,), num_neg, dtype=jnp.int32)

    partials = _make_sc_kernel(B)(
        gflat, target, context, jnp.asarray(packed), nn16)
    return jnp.sum(partials)


# speedup vs baseline: 1.0038x; 1.0038x over previous
import functools

import jax
import jax.numpy as jnp
import numpy as np
from jax import lax
from jax.experimental import pallas as pl
from jax.experimental.pallas import tpu as pltpu
from jax.experimental.pallas import tpu_sc as plsc

_VP = 1024
_L = 16
_NW = 32
_NNEG = 5
_GCH = 128


def _gram_body(ab_hbm, o_ref, a_v, b_v, sem):
    v = ab_hbm.shape[0] // 2
    ca = pltpu.make_async_copy(ab_hbm.at[pl.ds(0, v)], a_v, sem)
    cb = pltpu.make_async_copy(ab_hbm.at[pl.ds(v, v)], b_v, sem)
    ca.start()
    cb.start()
    ca.wait()
    cb.wait()
    a = jnp.pad(a_v[...], ((0, _VP - v), (0, 0)))
    b = jnp.pad(b_v[...], ((0, _VP - v), (0, 0)))
    g = lax.dot_general(a, b, (((1,), (1,)), ((), ())),
                        preferred_element_type=jnp.float32)
    o_ref[...] = g.reshape(_VP * _VP // 128, 128)


def _log_sigmoid(x):
    u = jnp.exp(-jnp.abs(x))
    s = u / (2.0 + u)
    s2 = s * s
    p = 1.0 + s2 * (1.0 / 3.0 + s2 * (1.0 / 5.0 + s2 * (1.0 / 7.0 + s2 * (1.0 / 9.0))))
    return jnp.minimum(x, 0.0) - 2.0 * s * p


def _rotl(x, d):
    d = np.uint32(d)
    return (x << d) | (x >> np.uint32(32 - d))


def _threefry2x32(k1, k2, x0, x1):
    rots = ((13, 15, 26, 6), (17, 29, 16, 24))
    ks = (np.uint32(k1), np.uint32(k2),
          np.uint32(k1) ^ np.uint32(k2) ^ np.uint32(0x1BD11BDA))
    x0 = (x0 + ks[0]).astype(np.uint32)
    x1 = (x1 + ks[1]).astype(np.uint32)
    for i in range(5):
        for r in rots[i % 2]:
            x0 = (x0 + x1).astype(np.uint32)
            x1 = x0 ^ _rotl(x1, r)
        x0 = (x0 + ks[(i + 1) % 3]).astype(np.uint32)
        x1 = (x1 + ks[(i + 2) % 3] + np.uint32(i + 1)).astype(np.uint32)
    return x0, x1


def _np_randint(k1, k2, n, span):
    hi2 = np.zeros(2, np.uint32)
    lo2 = np.arange(2, dtype=np.uint32)
    b1, b2 = _threefry2x32(k1, k2, hi2, lo2)
    idx = np.arange(n, dtype=np.uint64)
    chi = (idx >> np.uint64(32)).astype(np.uint32)
    clo = (idx & np.uint64(0xFFFFFFFF)).astype(np.uint32)
    h1, h2 = _threefry2x32(b1[0], b2[0], chi, clo)
    l1, l2 = _threefry2x32(b1[1], b2[1], chi, clo)
    higher, lower = h1 ^ h2, l1 ^ l2
    span = np.uint32(span)
    mult = np.uint32((int(2**16 % span) ** 2) % int(span))
    return (((higher % span) * mult + lower % span) % span).astype(np.int32)


@functools.cache
def _neg_draw(B, V):
    neg = _np_randint(np.uint32(0), np.uint32(42), B * _NNEG, V).reshape(B, _NNEG)
    f0, f1 = _threefry2x32(np.uint32(0), np.uint32(42),
                           np.zeros(1, np.uint32), np.ones(1, np.uint32))
    res = _np_randint(f0[0], f1[0], B * _NNEG, V).reshape(B, _NNEG)
    bpw = B // _NW
    neg_wj = neg.reshape(_NW, bpw, _NNEG).transpose(0, 2, 1).reshape(-1)
    res_wj = res.reshape(_NW, bpw, _NNEG).transpose(0, 2, 1).reshape(-1)
    return np.ascontiguousarray(neg_wj | (res_wj << 10)).astype(np.int32)


@functools.cache
def _make_sc_kernel(B):
    bpw = B // _NW
    npw = bpw * _NNEG
    chunks = bpw // _L
    scale = -1.0 / B
    mesh = plsc.VectorSubcoreMesh(core_axis_name="c", subcore_axis_name="s")

    @functools.partial(
        pl.kernel, mesh=mesh,
        out_type=jax.ShapeDtypeStruct((_NW, _L), jnp.float32),
        scratch_types=[
            pltpu.VMEM((bpw,), jnp.int32),
            pltpu.VMEM((bpw,), jnp.int32),
            pltpu.VMEM((npw,), jnp.int32),
            pltpu.VMEM((_L,), jnp.int32),
            pltpu.VMEM((bpw,), jnp.int32),
            pltpu.VMEM((npw,), jnp.int32),
            pltpu.VMEM((bpw,), jnp.float32),
            pltpu.VMEM((npw,), jnp.float32),
            pltpu.VMEM((_L,), jnp.float32),
            pltpu.SemaphoreType.DMA,
            pltpu.SemaphoreType.DMA,
            pltpu.SemaphoreType.DMA,
            pltpu.SemaphoreType.DMA,
        ] + [pltpu.SemaphoreType.DMA] * _NNEG,
    )
    def sc_kernel(g_hbm, tgt_hbm, ctx_hbm, pk_hbm, nn_hbm, out_hbm,
                  tgt_v, ctx_v, pk_v, nn_v, pidx_v, nidx_v,
                  ps_v, ns_v, acc_v, semi, semk, semn, semp, *semj):
        info = plsc.get_sparse_core_info()
        wid = lax.axis_index("c") * info.num_subcores + lax.axis_index("s")
        eb = wid * bpw
        nb = wid * npw
        ci_t = pltpu.async_copy(tgt_hbm.at[pl.ds(eb, bpw)], tgt_v, semi)
        ci_c = pltpu.async_copy(ctx_hbm.at[pl.ds(eb, bpw)], ctx_v, semi)
        ci_p = pltpu.async_copy(pk_hbm.at[pl.ds(nb, npw)], pk_v, semk)
        ci_n = pltpu.async_copy(nn_hbm, nn_v, semn)
        ci_t.wait()
        ci_c.wait()

        def pos_idx_body(c, _):
            t16 = tgt_v[pl.ds(c * _L, _L)]
            c16 = ctx_v[pl.ds(c * _L, _L)]
            pidx_v[pl.ds(c * _L, _L)] = t16 * _VP + c16
            return 0
        lax.fori_loop(0, chunks, pos_idx_body, 0)
        pos_handles = [
            pltpu.async_copy(g_hbm.at[pidx_v.at[pl.ds(k * _GCH, _GCH)]],
                             ps_v.at[pl.ds(k * _GCH, _GCH)], semp)
            for k in range(bpw // _GCH)]

        ci_p.wait()
        neg_handles = []
        for j in range(_NNEG):
            def neg_idx_body(c, _, j=j):
                t16 = tgt_v[pl.ds(c * _L, _L)]
                c16 = ctx_v[pl.ds(c * _L, _L)]
                w16 = pk_v[pl.ds(j * bpw + c * _L, _L)]
                n16 = lax.bitwise_and(w16, _VP - 1)
                r16 = lax.shift_right_logical(w16, 10)
                n16 = jnp.where(n16 == c16, r16, n16)
                nidx_v[pl.ds(j * bpw + c * _L, _L)] = t16 * _VP + n16
                return 0
            lax.fori_loop(0, chunks, neg_idx_body, 0)
            neg_handles.append([
                pltpu.async_copy(
                    g_hbm.at[nidx_v.at[pl.ds(j * bpw + k * _GCH, _GCH)]],
                    ns_v.at[pl.ds(j * bpw + k * _GCH, _GCH)], semj[j])
                for k in range(bpw // _GCH)])

        for h in pos_handles:
            h.wait()

        def pos_acc_body(c, acc):
            x = ps_v[pl.ds(c * _L, _L)]
            return acc + _log_sigmoid(x)
        acc = lax.fori_loop(0, chunks, pos_acc_body,
                            jnp.zeros((_L,), jnp.float32))

        ci_n.wait()
        nn16 = nn_v[...]
        for j in range(_NNEG):
            for h in neg_handles[j]:
                h.wait()

            def neg_acc_body(c, jacc, j=j):
                x = ns_v[pl.ds(j * bpw + c * _L, _L)]
                return jacc + _log_sigmoid(-x)
            jacc = lax.fori_loop(0, chunks, neg_acc_body,
                                 jnp.zeros((_L,), jnp.float32))
            acc = acc + jnp.where(nn16 > j, jacc, jnp.zeros((_L,), jnp.float32))

        acc_v[...] = acc * scale
        pltpu.sync_copy(acc_v, out_hbm.at[wid])

    return sc_kernel


def kernel(in_table, out_table, target, context, num_neg):
    V, _ = in_table.shape
    B = target.shape[0]
    ab = jnp.concatenate(
        [in_table.astype(jnp.bfloat16), out_table.astype(jnp.bfloat16)], 0)
    gram = pl.pallas_call(
        _gram_body,
        in_specs=[pl.BlockSpec(memory_space=pltpu.MemorySpace.HBM)],
        out_shape=jax.ShapeDtypeStruct((_VP * _VP // 128, 128), jnp.float32),
        scratch_shapes=[
            pltpu.VMEM(in_table.shape, jnp.bfloat16),
            pltpu.VMEM(out_table.shape, jnp.bfloat16),
            pltpu.SemaphoreType.DMA,
        ],
    )(ab)
    gflat = gram.reshape(_VP * _VP)

    packed = _neg_draw(B, V)
    nn16 = jnp.full((_L,), num_neg, dtype=jnp.int32)

    partials = _make_sc_kernel(B)(
        gflat, target, context, jnp.asarray(packed), nn16)
    return jnp.sum(partials)
